# baseline (speedup 1.0000x reference)
import functools

import jax
import jax.numpy as jnp
from jax import lax
from jax.experimental import pallas as pl
from jax.experimental.pallas import tpu as pltpu
from jax.experimental.pallas import tpu_sc as plsc

_N = 10000
_NPF = 69
_EMB = 128
_E = 320000
_ACCW = 144
_NC = 2
_NS = 16
_NWK = _NC * _NS
_EPW = _E // _NWK
_CB = 40
_NCH = _EPW // _CB
_CPB = 50
_NBLK = _NCH // _CPB
_PAIRS = _CPB // 2
_RPT = _N // _NS
_ZR = 25

_BR = 2000
_GRID = _N // _BR


def _relu(x):
    return jnp.maximum(x, 0.0)


def _dot(x, w):
    return jnp.dot(x, w, preferred_element_type=jnp.float32)


def _make_edge_pass():
    mesh = plsc.VectorSubcoreMesh(core_axis_name="c", subcore_axis_name="s")

    @functools.partial(
        pl.kernel,
        out_type=jax.ShapeDtypeStruct((_NC, _N, _ACCW), jnp.float32),
        mesh=mesh,
        compiler_params=pltpu.CompilerParams(use_tc_tiling_on_sc=False),
        scratch_types=[
            pltpu.VMEM((_CPB, _CB), jnp.int32),
            pltpu.VMEM((_CPB, _CB), jnp.int32),
            pltpu.VMEM((_CB, _EMB), jnp.float32),
            pltpu.VMEM((_CB, _EMB), jnp.float32),
            pltpu.VMEM((_CB, _EMB), jnp.float32),
            pltpu.VMEM((_CB, _EMB), jnp.float32),
            pltpu.VMEM((_CB, _ACCW), jnp.float32),
            pltpu.VMEM((_CB, _ACCW), jnp.float32),
            pltpu.VMEM((_ZR, _ACCW), jnp.float32),
            pltpu.VMEM_SHARED((_N, _ACCW), jnp.float32),
            pltpu.SemaphoreType.DMA,
            pltpu.SemaphoreType.DMA,
            pltpu.SemaphoreType.DMA,
            pltpu.SemaphoreType.DMA,
            pltpu.SemaphoreType.DMA,
            pltpu.SemaphoreType.DMA,
        ],
    )
    def edge_pass(a_hbm, b_hbm, src_hbm, dst_hbm, out_hbm,
                  sidx, didx, arow0, brow0, arow1, brow1, msg0, msg1,
                  zbuf, acc, sa0, sb0, sa1, sb1, ss0, ss1):
        cid = lax.axis_index("c")
        sid = lax.axis_index("s")
        wid = sid * _NC + cid

        zero16 = jnp.zeros((16,), jnp.float32)
        one_hot0 = jnp.where(lax.iota(jnp.int32, 16) == 0,
                             jnp.float32(1.0), jnp.float32(0.0))

        def zrow(r, carry):
            for j in range(_ACCW // 16):
                zbuf[r, pl.ds(j * 16, 16)] = zero16
            return carry
        lax.fori_loop(0, _ZR, zrow, 0)

        def mrow(r, carry):
            msg0[r, pl.ds(_EMB, 16)] = one_hot0
            msg1[r, pl.ds(_EMB, 16)] = one_hot0
            return carry
        lax.fori_loop(0, _CB, mrow, 0)

        for i in range(_RPT // _ZR):
            pltpu.sync_copy(zbuf, acc.at[pl.ds(sid * _RPT + i * _ZR, _ZR)])
        plsc.subcore_barrier()

        def gissue(j, ar, br, sa, sb):
            pltpu.async_copy(a_hbm.at[didx.at[j]], ar, sa)
            pltpu.async_copy(b_hbm.at[sidx.at[j]], br, sb)

        def gwait(ar, br, sa, sb):
            pltpu.make_async_copy(a_hbm.at[didx.at[0]], ar, sa).wait()
            pltpu.make_async_copy(b_hbm.at[sidx.at[0]], br, sb).wait()

        def sissue(j, mg, ss):
            pltpu.async_copy(mg, acc.at[didx.at[j]], ss, add=True)

        def swait(mg, ss):
            pltpu.make_async_copy(mg, acc.at[didx.at[0]], ss).wait()

        def compute(ar, br, mg):
            @plsc.parallel_loop(0, _CB, step=1, unroll=4)
            def row(r):
                for j in range(_EMB // 16):
                    s = pl.ds(j * 16, 16)
                    mg[r, s] = jnp.maximum(ar[r, s] + br[r, s], 0.0)

        for blk in range(_NBLK):
            r0 = wid * _NCH + blk * _CPB
            pltpu.sync_copy(src_hbm.at[pl.ds(r0, _CPB)], sidx)
            pltpu.sync_copy(dst_hbm.at[pl.ds(r0, _CPB)], didx)
            gissue(0, arow0, brow0, sa0, sb0)

            def pair(i, carry):
                j0 = 2 * i
                gissue(j0 + 1, arow1, brow1, sa1, sb1)
                gwait(arow0, brow0, sa0, sb0)

                @pl.when(i > 0)
                def _():
                    swait(msg0, ss0)
                compute(arow0, brow0, msg0)
                sissue(j0, msg0, ss0)

                @pl.when(i < _PAIRS - 1)
                def _():
                    gissue(j0 + 2, arow0, brow0, sa0, sb0)
                gwait(arow1, brow1, sa1, sb1)

                @pl.when(i > 0)
                def _():
                    swait(msg1, ss1)
                compute(arow1, brow1, msg1)
                sissue(j0 + 1, msg1, ss1)
                return carry
            lax.fori_loop(0, _PAIRS, pair, 0)
            swait(msg0, ss0)
            swait(msg1, ss1)

        plsc.subcore_barrier()
        r0 = sid * _RPT
        pltpu.sync_copy(acc.at[pl.ds(r0, _RPT)],
                        out_hbm.at[cid, pl.ds(r0, _RPT)])

    return edge_pass


_edge_pass_cache = []


def _edge_pass(a, b, src, dst):
    if not _edge_pass_cache:
        _edge_pass_cache.append(_make_edge_pass())
    return _edge_pass_cache[0](a, b, src, dst)


def _full(shape):
    return pl.BlockSpec(shape, lambda i: tuple(0 for _ in shape))


def _rows(width):
    return pl.BlockSpec((_BR, width), lambda i: (i, 0))


def _d1_body(pv_x, ch_x,
             pesh, pesc, peW1, peb1, peW2, peb2,
             cesh, cesc, ceW1, ceb1, ceW2, ceb2,
             fml1W, fml1b, fmr1W, fml2W, fml2b,
             pv0_o, ch0_o, a1_o, b1_o, a2_o):
    x = (pv_x[...] + pesh[...]) * pesc[...]
    h = _relu(_dot(x, peW1[...]) + peb1[...])
    pv0 = _relu(_dot(h, peW2[...]) + peb2[...])
    y = (ch_x[...] + cesh[...]) * cesc[...]
    g = _relu(_dot(y, ceW1[...]) + ceb1[...])
    ch0 = _relu(_dot(g, ceW2[...]) + ceb2[...])
    pv0_o[...] = pv0
    ch0_o[...] = ch0
    a1_o[...] = _dot(ch0, fml1W[...]) + fml1b[...]
    b1_o[...] = _dot(pv0, fmr1W[...])
    a2_o[...] = _dot(pv0, fml2W[...]) + fml2b[...]


def _agg(s_blk, finW, finb):
    t = s_blk[0] + s_blk[1]
    accb = t[:, :_EMB]
    cnt = t[:, _EMB:_EMB + 1]
    summed = _dot(accb, finW) + cnt * finb
    return summed / jnp.maximum(cnt, 1.0)


def _d2_body(s1, ch0, fin1W, fin1b, o1t1, o1o1, o1c1, o2W1, o2b1, fmr2W,
             b2_o):
    agg = _agg(s1[...], fin1W[...], fin1b[...])
    h = _relu(_dot(agg, o1t1[...]) + _dot(ch0[...], o1o1[...]) + o1c1[...])
    ch1 = _dot(h, o2W1[...]) + o2b1[...]
    b2_o[...] = _dot(ch1, fmr2W[...])


def _d3_body(s2, pv0, fin2W, fin2b, o1t2, o1o2, o1c2, o2W2, o2b2,
             outW1, outb1, outW2, outb2, out_o):
    agg = _agg(s2[...], fin2W[...], fin2b[...])
    h = _relu(_dot(agg, o1t2[...]) + _dot(pv0[...], o1o2[...]) + o1c2[...])
    pv1 = _dot(h, o2W2[...]) + o2b2[...]
    z = _relu(_dot(pv1, outW1[...]) + outb1[...])
    out_o[...] = jax.nn.sigmoid(_dot(z, outW2[...]) + outb2[...])


def _row2d(v):
    return v.reshape(1, -1)


def kernel(pivot_node_features, edge_indices, children_features, params):
    p = params
    pe, ce = p["pivot_emb"], p["child_emb"]
    c1, c2 = p["conv_v_to_c"], p["conv_c_to_v"]

    s1, s2 = c1["final_scale"][0], c2["final_scale"][0]
    p1, p2 = c1["post_scale"][0], c2["post_scale"][0]
    fml1W, fml1b = c1["fml_W"] * s1, _row2d(c1["fml_b"] * s1)
    fmr1W = c1["fmr_W"] * s1
    fml2W, fml2b = c2["fml_W"] * s2, _row2d(c2["fml_b"] * s2)
    fmr2W = c2["fmr_W"] * s2
    fin1W, fin1b = c1["fin_W"] * p1, _row2d(c1["fin_b"] * p1)
    fin2W, fin2b = c2["fin_W"] * p2, _row2d(c2["fin_b"] * p2)
    o1t1, o1o1 = c1["o1_W"][:_EMB], c1["o1_W"][_EMB:]
    o1t2, o1o2 = c2["o1_W"][:_EMB], c2["o1_W"][_EMB:]

    emb_shape = jax.ShapeDtypeStruct((_N, _EMB), jnp.float32)
    wfull = _full

    pv0, ch0, a1, b1, a2 = pl.pallas_call(
        _d1_body,
        grid=(_GRID,),
        in_specs=[
            _rows(_NPF), _rows(_NPF),
            wfull((1, _NPF)), wfull((1, _NPF)), wfull((_NPF, _EMB)),
            wfull((1, _EMB)), wfull((_EMB, _EMB)), wfull((1, _EMB)),
            wfull((1, _NPF)), wfull((1, _NPF)), wfull((_NPF, _EMB)),
            wfull((1, _EMB)), wfull((_EMB, _EMB)), wfull((1, _EMB)),
            wfull((_EMB, _EMB)), wfull((1, _EMB)), wfull((_EMB, _EMB)),
            wfull((_EMB, _EMB)), wfull((1, _EMB)),
        ],
        out_specs=[_rows(_EMB)] * 5,
        out_shape=[emb_shape] * 5,
    )(
        pivot_node_features, children_features,
        _row2d(pe["shift"]), _row2d(pe["scale"]), pe["W1"], _row2d(pe["b1"]),
        pe["W2"], _row2d(pe["b2"]),
        _row2d(ce["shift"]), _row2d(ce["scale"]), ce["W1"], _row2d(ce["b1"]),
        ce["W2"], _row2d(ce["b2"]),
        fml1W, fml1b, fmr1W, fml2W, fml2b,
    )

    src = edge_indices[0].reshape(_NWK * _NCH, _CB)
    dst = edge_indices[1].reshape(_NWK * _NCH, _CB)

    s1acc = _edge_pass(a1, b1, src, dst)

    sspec = pl.BlockSpec((_NC, _BR, _ACCW), lambda i: (0, i, 0))
    (b2,) = pl.pallas_call(
        _d2_body,
        grid=(_GRID,),
        in_specs=[
            sspec, _rows(_EMB),
            wfull((_EMB, _EMB)), wfull((1, _EMB)),
            wfull((_EMB, _EMB)), wfull((_EMB, _EMB)), wfull((1, _EMB)),
            wfull((_EMB, _EMB)), wfull((1, _EMB)), wfull((_EMB, _EMB)),
        ],
        out_specs=[_rows(_EMB)],
        out_shape=[emb_shape],
    )(
        s1acc, ch0,
        fin1W, fin1b, o1t1, o1o1, _row2d(c1["o1_b"]),
        c1["o2_W"], _row2d(c1["o2_b"]), fmr2W,
    )

    s2acc = _edge_pass(a2, b2, dst, src)

    (out,) = pl.pallas_call(
        _d3_body,
        grid=(_GRID,),
        in_specs=[
            sspec, _rows(_EMB),
            wfull((_EMB, _EMB)), wfull((1, _EMB)),
            wfull((_EMB, _EMB)), wfull((_EMB, _EMB)), wfull((1, _EMB)),
            wfull((_EMB, _EMB)), wfull((1, _EMB)),
            wfull((_EMB, _EMB)), wfull((1, _EMB)),
            wfull((_EMB, 2)), wfull((1, 2)),
        ],
        out_specs=[_rows(2)],
        out_shape=[jax.ShapeDtypeStruct((_N, 2), jnp.float32)],
    )(
        s2acc, pv0,
        fin2W, fin2b, o1t2, o1o2, _row2d(c2["o1_b"]),
        c2["o2_W"], _row2d(c2["o2_b"]),
        p["out_W1"], _row2d(p["out_b1"]), p["out_W2"], _row2d(p["out_b2"]),
    )
    return out


# speedup vs baseline: 1.1809x; 1.1809x over previous
import functools

import jax
import jax.numpy as jnp
from jax import lax
from jax.experimental import pallas as pl
from jax.experimental.pallas import tpu as pltpu
from jax.experimental.pallas import tpu_sc as plsc

_N = 10000
_NPF = 69
_EMB = 128
_E = 320000
_CNTW = 32
_NC = 2
_NS = 16
_NWK = _NC * _NS
_EPW = _E // _NWK
_CB = 40
_NCH = _EPW // _CB
_CPB = 50
_NBLK = _NCH // _CPB
_TRIPS = (_CPB - 2) // 3
_RPT = _N // _NS
_ZR = 5

_BR = 2000
_GRID = _N // _BR


def _relu(x):
    return jnp.maximum(x, 0.0)


def _dot(x, w):
    return jnp.dot(x, w, preferred_element_type=jnp.float32)


def _make_count_pass():
    mesh = plsc.VectorSubcoreMesh(core_axis_name="c", subcore_axis_name="s")

    @functools.partial(
        pl.kernel,
        out_type=jax.ShapeDtypeStruct((_NC, _N, _CNTW), jnp.float32),
        mesh=mesh,
        compiler_params=pltpu.CompilerParams(use_tc_tiling_on_sc=False),
        scratch_types=[
            pltpu.VMEM((_CPB, _CB), jnp.int32),
            pltpu.VMEM((_CPB, _CB), jnp.int32),
            pltpu.VMEM((_CB, _CNTW), jnp.float32),
            pltpu.VMEM((_CB, _CNTW), jnp.float32),
            pltpu.VMEM((_ZR, _CNTW), jnp.float32),
            pltpu.VMEM_SHARED((_N, _CNTW), jnp.float32),
            pltpu.SemaphoreType.DMA,
            pltpu.SemaphoreType.DMA,
        ],
    )
    def count_pass(src_hbm, dst_hbm, out_hbm,
                   sidx, didx, oned, ones, zbuf, acc, ssc, sz):
        cid = lax.axis_index("c")
        sid = lax.axis_index("s")
        wid = sid * _NC + cid
        zero16 = jnp.zeros((16,), jnp.float32)
        one_hot0 = jnp.where(lax.iota(jnp.int32, 16) == 0,
                             jnp.float32(1.0), jnp.float32(0.0))

        def irow(r, carry):
            oned[r, pl.ds(0, 16)] = one_hot0
            oned[r, pl.ds(16, 16)] = zero16
            ones[r, pl.ds(0, 16)] = zero16
            ones[r, pl.ds(16, 16)] = one_hot0
            return carry
        lax.fori_loop(0, _CB, irow, 0)

        def zrow(r, carry):
            zbuf[r, pl.ds(0, 16)] = zero16
            zbuf[r, pl.ds(16, 16)] = zero16
            return carry
        lax.fori_loop(0, _ZR, zrow, 0)

        def zcopy(i, carry):
            pltpu.async_copy(zbuf, acc.at[pl.ds(sid * _RPT + i * _ZR, _ZR)],
                             sz)
            return carry
        lax.fori_loop(0, _RPT // _ZR, zcopy, 0)

        def zdrain(i, carry):
            pltpu.make_async_copy(zbuf, acc.at[pl.ds(0, _ZR)], sz).wait()
            return carry
        lax.fori_loop(0, _RPT // _ZR, zdrain, 0)
        plsc.subcore_barrier()

        for blk in range(_NBLK):
            r0 = wid * _NCH + blk * _CPB
            pltpu.sync_copy(src_hbm.at[pl.ds(r0, _CPB)], sidx)
            pltpu.sync_copy(dst_hbm.at[pl.ds(r0, _CPB)], didx)

            def fire(c, carry):
                pltpu.async_copy(oned, acc.at[didx.at[c]], ssc, add=True)
                pltpu.async_copy(ones, acc.at[sidx.at[c]], ssc, add=True)
                return carry
            lax.fori_loop(0, _CPB, fire, 0)

            def drain(c, carry):
                pltpu.make_async_copy(oned, acc.at[didx.at[0]], ssc).wait()
                pltpu.make_async_copy(ones, acc.at[sidx.at[0]], ssc).wait()
                return carry
            lax.fori_loop(0, _CPB, drain, 0)

        plsc.subcore_barrier()
        r0 = sid * _RPT
        pltpu.sync_copy(acc.at[pl.ds(r0, _RPT)],
                        out_hbm.at[cid, pl.ds(r0, _RPT)])

    return count_pass


def _make_edge_pass():
    mesh = plsc.VectorSubcoreMesh(core_axis_name="c", subcore_axis_name="s")

    @functools.partial(
        pl.kernel,
        out_type=jax.ShapeDtypeStruct((_NC, _N, _EMB), jnp.float32),
        mesh=mesh,
        compiler_params=pltpu.CompilerParams(use_tc_tiling_on_sc=False),
        scratch_types=[
            pltpu.VMEM((_CPB, _CB), jnp.int32),
            pltpu.VMEM((_CPB, _CB), jnp.int32),
            pltpu.VMEM((_CB, _EMB), jnp.float32),
            pltpu.VMEM((_CB, _EMB), jnp.float32),
            pltpu.VMEM((_CB, _EMB), jnp.float32),
            pltpu.VMEM((_CB, _EMB), jnp.float32),
            pltpu.VMEM((_CB, _EMB), jnp.float32),
            pltpu.VMEM((_CB, _EMB), jnp.float32),
            pltpu.VMEM((_CB, _EMB), jnp.float32),
            pltpu.VMEM((_CB, _EMB), jnp.float32),
            pltpu.VMEM((_CB, _EMB), jnp.float32),
            pltpu.VMEM((_ZR, _EMB), jnp.float32),
            pltpu.VMEM_SHARED((_N, _EMB), jnp.float32),
            pltpu.SemaphoreType.DMA,
            pltpu.SemaphoreType.DMA,
            pltpu.SemaphoreType.DMA,
            pltpu.SemaphoreType.DMA,
            pltpu.SemaphoreType.DMA,
            pltpu.SemaphoreType.DMA,
            pltpu.SemaphoreType.DMA,
            pltpu.SemaphoreType.DMA,
            pltpu.SemaphoreType.DMA,
            pltpu.SemaphoreType.DMA,
        ],
    )
    def edge_pass(a_hbm, b_hbm, src_hbm, dst_hbm, out_hbm,
                  sidx, didx, a0, b0, a1, b1, a2, b2, m0, m1, m2,
                  zbuf, acc, sa0, sb0, sa1, sb1, sa2, sb2,
                  ss0, ss1, ss2, sz):
        cid = lax.axis_index("c")
        sid = lax.axis_index("s")
        wid = sid * _NC + cid
        ab = ((a0, b0, sa0, sb0), (a1, b1, sa1, sb1), (a2, b2, sa2, sb2))
        ms = ((m0, ss0), (m1, ss1), (m2, ss2))

        zero16 = jnp.zeros((16,), jnp.float32)

        def zrow(r, carry):
            for j in range(_EMB // 16):
                zbuf[r, pl.ds(j * 16, 16)] = zero16
            return carry
        lax.fori_loop(0, _ZR, zrow, 0)

        def zcopy(i, carry):
            pltpu.async_copy(zbuf, acc.at[pl.ds(sid * _RPT + i * _ZR, _ZR)],
                             sz)
            return carry
        lax.fori_loop(0, _RPT // _ZR, zcopy, 0)

        def zdrain(i, carry):
            pltpu.make_async_copy(zbuf, acc.at[pl.ds(0, _ZR)], sz).wait()
            return carry
        lax.fori_loop(0, _RPT // _ZR, zdrain, 0)
        plsc.subcore_barrier()

        def gissue(j, p):
            ar, br, sa, sb = ab[p]
            pltpu.async_copy(a_hbm.at[didx.at[j]], ar, sa)
            pltpu.async_copy(b_hbm.at[sidx.at[j]], br, sb)

        def gwait(p):
            ar, br, sa, sb = ab[p]
            pltpu.make_async_copy(a_hbm.at[didx.at[0]], ar, sa).wait()
            pltpu.make_async_copy(b_hbm.at[sidx.at[0]], br, sb).wait()

        def sissue(j, p):
            mg, ss = ms[p]
            pltpu.async_copy(mg, acc.at[didx.at[j]], ss, add=True)

        def swait(p):
            mg, ss = ms[p]
            pltpu.make_async_copy(mg, acc.at[didx.at[0]], ss).wait()

        def compute(p):
            ar, br, _, _ = ab[p]
            mg, _ = ms[p]

            @plsc.parallel_loop(0, _CB, step=1, unroll=2)
            def row(r):
                for j in range(_EMB // 16):
                    s = pl.ds(j * 16, 16)
                    mg[r, s] = jnp.maximum(ar[r, s] + br[r, s], 0.0)

        for blk in range(_NBLK):
            r0 = wid * _NCH + blk * _CPB
            pltpu.sync_copy(src_hbm.at[pl.ds(r0, _CPB)], sidx)
            pltpu.sync_copy(dst_hbm.at[pl.ds(r0, _CPB)], didx)
            gissue(0, 0)
            gissue(1, 1)

            def step(i, carry):
                for k in range(3):
                    c = 3 * i + k
                    gissue(c + 2, (k + 2) % 3)
                    gwait(k)

                    @pl.when(i > 0)
                    def _():
                        swait(k)
                    compute(k)
                    sissue(c, k)
                return carry
            lax.fori_loop(0, _TRIPS, step, 0)

            for c, p in ((_CPB - 2, 0), (_CPB - 1, 1)):
                gwait(p)
                swait(p)
                compute(p)
                sissue(c, p)
            swait(2)
            swait(0)
            swait(1)

        plsc.subcore_barrier()
        r0 = sid * _RPT
        pltpu.sync_copy(acc.at[pl.ds(r0, _RPT)],
                        out_hbm.at[cid, pl.ds(r0, _RPT)])

    return edge_pass


_edge_pass_cache = []


def _edge_pass(a, b, src, dst):
    if not _edge_pass_cache:
        _edge_pass_cache.append(_make_edge_pass())
    return _edge_pass_cache[0](a, b, src, dst)


_count_pass_cache = []


def _count_pass(src, dst):
    if not _count_pass_cache:
        _count_pass_cache.append(_make_count_pass())
    return _count_pass_cache[0](src, dst)


def _full(shape):
    return pl.BlockSpec(shape, lambda i: tuple(0 for _ in shape))


def _rows(width):
    return pl.BlockSpec((_BR, width), lambda i: (i, 0))


def _d1_body(pv_x, ch_x,
             pesh, pesc, peW1, peb1, peW2, peb2,
             cesh, cesc, ceW1, ceb1, ceW2, ceb2,
             fml1W, fml1b, fmr1W, fml2W, fml2b,
             pv0_o, ch0_o, a1_o, b1_o, a2_o):
    x = (pv_x[...] + pesh[...]) * pesc[...]
    h = _relu(_dot(x, peW1[...]) + peb1[...])
    pv0 = _relu(_dot(h, peW2[...]) + peb2[...])
    y = (ch_x[...] + cesh[...]) * cesc[...]
    g = _relu(_dot(y, ceW1[...]) + ceb1[...])
    ch0 = _relu(_dot(g, ceW2[...]) + ceb2[...])
    pv0_o[...] = pv0
    ch0_o[...] = ch0
    a1_o[...] = _dot(ch0, fml1W[...]) + fml1b[...]
    b1_o[...] = _dot(pv0, fmr1W[...])
    a2_o[...] = _dot(pv0, fml2W[...]) + fml2b[...]


def _agg(s_blk, c_blk, col, finW, finb):
    t = s_blk[0] + s_blk[1]
    cnt = c_blk[0, :, col:col + 1] + c_blk[1, :, col:col + 1]
    summed = _dot(t, finW) + cnt * finb
    return summed / jnp.maximum(cnt, 1.0)


def _d2_body(s1, cnt, ch0, fin1W, fin1b, o1t1, o1o1, o1c1, o2W1, o2b1, fmr2W,
             b2_o):
    agg = _agg(s1[...], cnt[...], 0, fin1W[...], fin1b[...])
    h = _relu(_dot(agg, o1t1[...]) + _dot(ch0[...], o1o1[...]) + o1c1[...])
    ch1 = _dot(h, o2W1[...]) + o2b1[...]
    b2_o[...] = _dot(ch1, fmr2W[...])


def _d3_body(s2, cnt, pv0, fin2W, fin2b, o1t2, o1o2, o1c2, o2W2, o2b2,
             outW1, outb1, outW2, outb2, out_o):
    agg = _agg(s2[...], cnt[...], 16, fin2W[...], fin2b[...])
    h = _relu(_dot(agg, o1t2[...]) + _dot(pv0[...], o1o2[...]) + o1c2[...])
    pv1 = _dot(h, o2W2[...]) + o2b2[...]
    z = _relu(_dot(pv1, outW1[...]) + outb1[...])
    out_o[...] = jax.nn.sigmoid(_dot(z, outW2[...]) + outb2[...])


def _row2d(v):
    return v.reshape(1, -1)


def kernel(pivot_node_features, edge_indices, children_features, params):
    p = params
    pe, ce = p["pivot_emb"], p["child_emb"]
    c1, c2 = p["conv_v_to_c"], p["conv_c_to_v"]

    s1, s2 = c1["final_scale"][0], c2["final_scale"][0]
    p1, p2 = c1["post_scale"][0], c2["post_scale"][0]
    fml1W, fml1b = c1["fml_W"] * s1, _row2d(c1["fml_b"] * s1)
    fmr1W = c1["fmr_W"] * s1
    fml2W, fml2b = c2["fml_W"] * s2, _row2d(c2["fml_b"] * s2)
    fmr2W = c2["fmr_W"] * s2
    fin1W, fin1b = c1["fin_W"] * p1, _row2d(c1["fin_b"] * p1)
    fin2W, fin2b = c2["fin_W"] * p2, _row2d(c2["fin_b"] * p2)
    o1t1, o1o1 = c1["o1_W"][:_EMB], c1["o1_W"][_EMB:]
    o1t2, o1o2 = c2["o1_W"][:_EMB], c2["o1_W"][_EMB:]

    emb_shape = jax.ShapeDtypeStruct((_N, _EMB), jnp.float32)
    wfull = _full

    pv0, ch0, a1, b1, a2 = pl.pallas_call(
        _d1_body,
        grid=(_GRID,),
        in_specs=[
            _rows(_NPF), _rows(_NPF),
            wfull((1, _NPF)), wfull((1, _NPF)), wfull((_NPF, _EMB)),
            wfull((1, _EMB)), wfull((_EMB, _EMB)), wfull((1, _EMB)),
            wfull((1, _NPF)), wfull((1, _NPF)), wfull((_NPF, _EMB)),
            wfull((1, _EMB)), wfull((_EMB, _EMB)), wfull((1, _EMB)),
            wfull((_EMB, _EMB)), wfull((1, _EMB)), wfull((_EMB, _EMB)),
            wfull((_EMB, _EMB)), wfull((1, _EMB)),
        ],
        out_specs=[_rows(_EMB)] * 5,
        out_shape=[emb_shape] * 5,
    )(
        pivot_node_features, children_features,
        _row2d(pe["shift"]), _row2d(pe["scale"]), pe["W1"], _row2d(pe["b1"]),
        pe["W2"], _row2d(pe["b2"]),
        _row2d(ce["shift"]), _row2d(ce["scale"]), ce["W1"], _row2d(ce["b1"]),
        ce["W2"], _row2d(ce["b2"]),
        fml1W, fml1b, fmr1W, fml2W, fml2b,
    )

    src = edge_indices[0].reshape(_NWK * _NCH, _CB)
    dst = edge_indices[1].reshape(_NWK * _NCH, _CB)

    cnt = _count_pass(src, dst)
    s1acc = _edge_pass(a1, b1, src, dst)

    sspec = pl.BlockSpec((_NC, _BR, _EMB), lambda i: (0, i, 0))
    cspec = pl.BlockSpec((_NC, _BR, _CNTW), lambda i: (0, i, 0))
    (b2,) = pl.pallas_call(
        _d2_body,
        grid=(_GRID,),
        in_specs=[
            sspec, cspec, _rows(_EMB),
            wfull((_EMB, _EMB)), wfull((1, _EMB)),
            wfull((_EMB, _EMB)), wfull((_EMB, _EMB)), wfull((1, _EMB)),
            wfull((_EMB, _EMB)), wfull((1, _EMB)), wfull((_EMB, _EMB)),
        ],
        out_specs=[_rows(_EMB)],
        out_shape=[emb_shape],
    )(
        s1acc, cnt, ch0,
        fin1W, fin1b, o1t1, o1o1, _row2d(c1["o1_b"]),
        c1["o2_W"], _row2d(c1["o2_b"]), fmr2W,
    )

    s2acc = _edge_pass(a2, b2, dst, src)

    (out,) = pl.pallas_call(
        _d3_body,
        grid=(_GRID,),
        in_specs=[
            sspec, cspec, _rows(_EMB),
            wfull((_EMB, _EMB)), wfull((1, _EMB)),
            wfull((_EMB, _EMB)), wfull((_EMB, _EMB)), wfull((1, _EMB)),
            wfull((_EMB, _EMB)), wfull((1, _EMB)),
            wfull((_EMB, _EMB)), wfull((1, _EMB)),
            wfull((_EMB, 2)), wfull((1, 2)),
        ],
        out_specs=[_rows(2)],
        out_shape=[jax.ShapeDtypeStruct((_N, 2), jnp.float32)],
    )(
        s2acc, cnt, pv0,
        fin2W, fin2b, o1t2, o1o2, _row2d(c2["o1_b"]),
        c2["o2_W"], _row2d(c2["o2_b"]),
        p["out_W1"], _row2d(p["out_b1"]), p["out_W2"], _row2d(p["out_b2"]),
    )
    return out


# speedup vs baseline: 1.5078x; 1.2768x over previous
import functools

import numpy as np

import jax
import jax.numpy as jnp
from jax import lax
from jax.experimental import pallas as pl
from jax.experimental.pallas import tpu as pltpu
from jax.experimental.pallas import tpu_sc as plsc

_N = 10000
_NPF = 69
_EMB = 128
_E = 320000
_CNTW = 32
_NC = 2
_NS = 16
_NWK = _NC * _NS
_EPW = _E // _NWK
_CB = 40
_NCH = _EPW // _CB
_CPB = 50
_NBLK = _NCH // _CPB
_TRIPS = (_CPB - 2) // 3
_RPT = _N // _NS
_ZR = 5

_BR = 2000
_GRID = _N // _BR

_PERM = np.empty((_EMB,), np.int32)
for _s in range(_EMB // 32):
    for _k in range(16):
        _PERM[32 * _s + 2 * _k] = 32 * _s + _k
        _PERM[32 * _s + 2 * _k + 1] = 32 * _s + 16 + _k


def _relu(x):
    return jnp.maximum(x, 0.0)


def _dot(x, w):
    return jnp.dot(x, w, preferred_element_type=jnp.float32)


def _make_count_pass():
    mesh = plsc.VectorSubcoreMesh(core_axis_name="c", subcore_axis_name="s")

    @functools.partial(
        pl.kernel,
        out_type=jax.ShapeDtypeStruct((_NC, _N, _CNTW), jnp.float32),
        mesh=mesh,
        compiler_params=pltpu.CompilerParams(use_tc_tiling_on_sc=False),
        scratch_types=[
            pltpu.VMEM((_CPB, _CB), jnp.int32),
            pltpu.VMEM((_CPB, _CB), jnp.int32),
            pltpu.VMEM((_CB, _CNTW), jnp.float32),
            pltpu.VMEM((_CB, _CNTW), jnp.float32),
            pltpu.VMEM((_ZR, _CNTW), jnp.float32),
            pltpu.VMEM_SHARED((_N, _CNTW), jnp.float32),
            pltpu.SemaphoreType.DMA,
            pltpu.SemaphoreType.DMA,
        ],
    )
    def count_pass(src_hbm, dst_hbm, out_hbm,
                   sidx, didx, oned, ones, zbuf, acc, ssc, sz):
        cid = lax.axis_index("c")
        sid = lax.axis_index("s")
        wid = sid * _NC + cid
        zero16 = jnp.zeros((16,), jnp.float32)
        one_hot0 = jnp.where(lax.iota(jnp.int32, 16) == 0,
                             jnp.float32(1.0), jnp.float32(0.0))

        def irow(r, carry):
            oned[r, pl.ds(0, 16)] = one_hot0
            oned[r, pl.ds(16, 16)] = zero16
            ones[r, pl.ds(0, 16)] = zero16
            ones[r, pl.ds(16, 16)] = one_hot0
            return carry
        lax.fori_loop(0, _CB, irow, 0)

        def zrow(r, carry):
            zbuf[r, pl.ds(0, 16)] = zero16
            zbuf[r, pl.ds(16, 16)] = zero16
            return carry
        lax.fori_loop(0, _ZR, zrow, 0)

        def zcopy(i, carry):
            pltpu.async_copy(zbuf, acc.at[pl.ds(sid * _RPT + i * _ZR, _ZR)],
                             sz)
            return carry
        lax.fori_loop(0, _RPT // _ZR, zcopy, 0)

        def zdrain(i, carry):
            pltpu.make_async_copy(zbuf, acc.at[pl.ds(0, _ZR)], sz).wait()
            return carry
        lax.fori_loop(0, _RPT // _ZR, zdrain, 0)
        plsc.subcore_barrier()

        for blk in range(_NBLK):
            r0 = wid * _NCH + blk * _CPB
            pltpu.sync_copy(src_hbm.at[pl.ds(r0, _CPB)], sidx)
            pltpu.sync_copy(dst_hbm.at[pl.ds(r0, _CPB)], didx)

            def fire(c, carry):
                pltpu.async_copy(oned, acc.at[didx.at[c]], ssc, add=True)
                pltpu.async_copy(ones, acc.at[sidx.at[c]], ssc, add=True)
                return carry
            lax.fori_loop(0, _CPB, fire, 0)

            def drain(c, carry):
                pltpu.make_async_copy(oned, acc.at[didx.at[0]], ssc).wait()
                pltpu.make_async_copy(ones, acc.at[sidx.at[0]], ssc).wait()
                return carry
            lax.fori_loop(0, _CPB, drain, 0)

        plsc.subcore_barrier()
        r0 = sid * _RPT
        pltpu.sync_copy(acc.at[pl.ds(r0, _RPT)],
                        out_hbm.at[cid, pl.ds(r0, _RPT)])

    return count_pass


def _make_edge_pass():
    mesh = plsc.VectorSubcoreMesh(core_axis_name="c", subcore_axis_name="s")

    @functools.partial(
        pl.kernel,
        out_type=jax.ShapeDtypeStruct((_NC, _N, _EMB), jnp.float32),
        mesh=mesh,
        compiler_params=pltpu.CompilerParams(use_tc_tiling_on_sc=False,
                                             needs_layout_passes=False),
        scratch_types=[
            pltpu.VMEM((_CPB, _CB), jnp.int32),
            pltpu.VMEM((_CPB, _CB), jnp.int32),
            pltpu.VMEM((_CB, _EMB), jnp.bfloat16),
            pltpu.VMEM((_CB, _EMB), jnp.bfloat16),
            pltpu.VMEM((_CB, _EMB), jnp.bfloat16),
            pltpu.VMEM((_CB, _EMB), jnp.bfloat16),
            pltpu.VMEM((_CB, _EMB), jnp.bfloat16),
            pltpu.VMEM((_CB, _EMB), jnp.bfloat16),
            pltpu.VMEM((_CB, _EMB), jnp.float32),
            pltpu.VMEM((_CB, _EMB), jnp.float32),
            pltpu.VMEM((_CB, _EMB), jnp.float32),
            pltpu.VMEM((_ZR, _EMB), jnp.float32),
            pltpu.VMEM_SHARED((_N, _EMB), jnp.float32),
            pltpu.SemaphoreType.DMA,
            pltpu.SemaphoreType.DMA,
            pltpu.SemaphoreType.DMA,
            pltpu.SemaphoreType.DMA,
            pltpu.SemaphoreType.DMA,
            pltpu.SemaphoreType.DMA,
            pltpu.SemaphoreType.DMA,
            pltpu.SemaphoreType.DMA,
            pltpu.SemaphoreType.DMA,
            pltpu.SemaphoreType.DMA,
        ],
    )
    def edge_pass(a_hbm, b_hbm, src_hbm, dst_hbm, out_hbm,
                  sidx, didx, a0, b0, a1, b1, a2, b2, m0, m1, m2,
                  zbuf, acc, sa0, sb0, sa1, sb1, sa2, sb2,
                  ss0, ss1, ss2, sz):
        cid = lax.axis_index("c")
        sid = lax.axis_index("s")
        wid = sid * _NC + cid
        ab = ((a0, b0, sa0, sb0), (a1, b1, sa1, sb1), (a2, b2, sa2, sb2))
        ms = ((m0, ss0), (m1, ss1), (m2, ss2))

        zero16 = jnp.zeros((16,), jnp.float32)

        def zrow(r, carry):
            for j in range(_EMB // 16):
                zbuf[r, pl.ds(j * 16, 16)] = zero16
            return carry
        lax.fori_loop(0, _ZR, zrow, 0)

        def zcopy(i, carry):
            pltpu.async_copy(zbuf, acc.at[pl.ds(sid * _RPT + i * _ZR, _ZR)],
                             sz)
            return carry
        lax.fori_loop(0, _RPT // _ZR, zcopy, 0)

        def zdrain(i, carry):
            pltpu.make_async_copy(zbuf, acc.at[pl.ds(0, _ZR)], sz).wait()
            return carry
        lax.fori_loop(0, _RPT // _ZR, zdrain, 0)
        plsc.subcore_barrier()

        def gissue(j, p):
            ar, br, sa, sb = ab[p]
            pltpu.async_copy(a_hbm.at[didx.at[j]], ar, sa)
            pltpu.async_copy(b_hbm.at[sidx.at[j]], br, sb)

        def gwait(p):
            ar, br, sa, sb = ab[p]
            pltpu.make_async_copy(a_hbm.at[didx.at[0]], ar, sa).wait()
            pltpu.make_async_copy(b_hbm.at[sidx.at[0]], br, sb).wait()

        def sissue(j, p):
            mg, ss = ms[p]
            pltpu.async_copy(mg, acc.at[didx.at[j]], ss, add=True)

        def swait(p):
            mg, ss = ms[p]
            pltpu.make_async_copy(mg, acc.at[didx.at[0]], ss).wait()

        def compute(p):
            ar, br, _, _ = ab[p]
            mg, _ = ms[p]

            @plsc.parallel_loop(0, _CB, step=1, unroll=2)
            def row(r):
                for j in range(_EMB // 32):
                    s = pl.ds(j * 32, 32)
                    v = jnp.maximum(ar[r, s] + br[r, s],
                                    jnp.bfloat16(0.0))
                    u0, u1 = plsc.unpack(v,
                                         format=plsc.PackFormat.INTERLEAVED)
                    mg[r, pl.ds(j * 32, 16)] = u0
                    mg[r, pl.ds(j * 32 + 16, 16)] = u1

        for blk in range(_NBLK):
            r0 = wid * _NCH + blk * _CPB
            pltpu.sync_copy(src_hbm.at[pl.ds(r0, _CPB)], sidx)
            pltpu.sync_copy(dst_hbm.at[pl.ds(r0, _CPB)], didx)
            gissue(0, 0)
            gissue(1, 1)

            def step(i, carry):
                for k in range(3):
                    c = 3 * i + k
                    gissue(c + 2, (k + 2) % 3)
                    gwait(k)

                    @pl.when(i > 0)
                    def _():
                        swait(k)
                    compute(k)
                    sissue(c, k)
                return carry
            lax.fori_loop(0, _TRIPS, step, 0)

            for c, p in ((_CPB - 2, 0), (_CPB - 1, 1)):
                gwait(p)
                swait(p)
                compute(p)
                sissue(c, p)
            swait(2)
            swait(0)
            swait(1)

        plsc.subcore_barrier()
        r0 = sid * _RPT
        pltpu.sync_copy(acc.at[pl.ds(r0, _RPT)],
                        out_hbm.at[cid, pl.ds(r0, _RPT)])

    return edge_pass


_edge_pass_cache = []


def _edge_pass(a, b, src, dst):
    if not _edge_pass_cache:
        _edge_pass_cache.append(_make_edge_pass())
    return _edge_pass_cache[0](a, b, src, dst)


_count_pass_cache = []


def _count_pass(src, dst):
    if not _count_pass_cache:
        _count_pass_cache.append(_make_count_pass())
    return _count_pass_cache[0](src, dst)


def _full(shape):
    return pl.BlockSpec(shape, lambda i: tuple(0 for _ in shape))


def _rows(width):
    return pl.BlockSpec((_BR, width), lambda i: (i, 0))


def _d1_body(pv_x, ch_x,
             pesh, pesc, peW1, peb1, peW2, peb2,
             cesh, cesc, ceW1, ceb1, ceW2, ceb2,
             fml1W, fml1b, fmr1W, fml2W, fml2b,
             pv0_o, ch0_o, a1_o, b1_o, a2_o):
    x = (pv_x[...] + pesh[...]) * pesc[...]
    h = _relu(_dot(x, peW1[...]) + peb1[...])
    pv0 = _relu(_dot(h, peW2[...]) + peb2[...])
    y = (ch_x[...] + cesh[...]) * cesc[...]
    g = _relu(_dot(y, ceW1[...]) + ceb1[...])
    ch0 = _relu(_dot(g, ceW2[...]) + ceb2[...])
    pv0_o[...] = pv0
    ch0_o[...] = ch0
    a1_o[...] = (_dot(ch0, fml1W[...]) + fml1b[...]).astype(jnp.bfloat16)
    b1_o[...] = _dot(pv0, fmr1W[...]).astype(jnp.bfloat16)
    a2_o[...] = (_dot(pv0, fml2W[...]) + fml2b[...]).astype(jnp.bfloat16)


def _agg(s_blk, c_blk, col, finW, finb):
    t = s_blk[0] + s_blk[1]
    cnt = c_blk[0, :, col:col + 1] + c_blk[1, :, col:col + 1]
    summed = _dot(t, finW) + cnt * finb
    return summed / jnp.maximum(cnt, 1.0)


def _d2_body(s1, cnt, ch0, fin1W, fin1b, o1t1, o1o1, o1c1, o2W1, o2b1, fmr2W,
             b2_o):
    agg = _agg(s1[...], cnt[...], 0, fin1W[...], fin1b[...])
    h = _relu(_dot(agg, o1t1[...]) + _dot(ch0[...], o1o1[...]) + o1c1[...])
    ch1 = _dot(h, o2W1[...]) + o2b1[...]
    b2_o[...] = _dot(ch1, fmr2W[...]).astype(jnp.bfloat16)


def _d3_body(s2, cnt, pv0, fin2W, fin2b, o1t2, o1o2, o1c2, o2W2, o2b2,
             outW1, outb1, outW2, outb2, out_o):
    agg = _agg(s2[...], cnt[...], 16, fin2W[...], fin2b[...])
    h = _relu(_dot(agg, o1t2[...]) + _dot(pv0[...], o1o2[...]) + o1c2[...])
    pv1 = _dot(h, o2W2[...]) + o2b2[...]
    z = _relu(_dot(pv1, outW1[...]) + outb1[...])
    out_o[...] = jax.nn.sigmoid(_dot(z, outW2[...]) + outb2[...])


def _row2d(v):
    return v.reshape(1, -1)


def kernel(pivot_node_features, edge_indices, children_features, params):
    p = params
    pe, ce = p["pivot_emb"], p["child_emb"]
    c1, c2 = p["conv_v_to_c"], p["conv_c_to_v"]

    s1, s2 = c1["final_scale"][0], c2["final_scale"][0]
    p1, p2 = c1["post_scale"][0], c2["post_scale"][0]
    fml1W = (c1["fml_W"] * s1)[:, _PERM]
    fml1b = _row2d((c1["fml_b"] * s1)[_PERM])
    fmr1W = (c1["fmr_W"] * s1)[:, _PERM]
    fml2W = (c2["fml_W"] * s2)[:, _PERM]
    fml2b = _row2d((c2["fml_b"] * s2)[_PERM])
    fmr2W = (c2["fmr_W"] * s2)[:, _PERM]
    fin1W, fin1b = c1["fin_W"] * p1, _row2d(c1["fin_b"] * p1)
    fin2W, fin2b = c2["fin_W"] * p2, _row2d(c2["fin_b"] * p2)
    o1t1, o1o1 = c1["o1_W"][:_EMB], c1["o1_W"][_EMB:]
    o1t2, o1o2 = c2["o1_W"][:_EMB], c2["o1_W"][_EMB:]

    emb_shape = jax.ShapeDtypeStruct((_N, _EMB), jnp.float32)
    bf_shape = jax.ShapeDtypeStruct((_N, _EMB), jnp.bfloat16)
    wfull = _full

    pv0, ch0, a1, b1, a2 = pl.pallas_call(
        _d1_body,
        grid=(_GRID,),
        in_specs=[
            _rows(_NPF), _rows(_NPF),
            wfull((1, _NPF)), wfull((1, _NPF)), wfull((_NPF, _EMB)),
            wfull((1, _EMB)), wfull((_EMB, _EMB)), wfull((1, _EMB)),
            wfull((1, _NPF)), wfull((1, _NPF)), wfull((_NPF, _EMB)),
            wfull((1, _EMB)), wfull((_EMB, _EMB)), wfull((1, _EMB)),
            wfull((_EMB, _EMB)), wfull((1, _EMB)), wfull((_EMB, _EMB)),
            wfull((_EMB, _EMB)), wfull((1, _EMB)),
        ],
        out_specs=[_rows(_EMB)] * 5,
        out_shape=[emb_shape, emb_shape, bf_shape, bf_shape, bf_shape],
    )(
        pivot_node_features, children_features,
        _row2d(pe["shift"]), _row2d(pe["scale"]), pe["W1"], _row2d(pe["b1"]),
        pe["W2"], _row2d(pe["b2"]),
        _row2d(ce["shift"]), _row2d(ce["scale"]), ce["W1"], _row2d(ce["b1"]),
        ce["W2"], _row2d(ce["b2"]),
        fml1W, fml1b, fmr1W, fml2W, fml2b,
    )

    src = edge_indices[0].reshape(_NWK * _NCH, _CB)
    dst = edge_indices[1].reshape(_NWK * _NCH, _CB)

    cnt = _count_pass(src, dst)
    s1acc = _edge_pass(a1, b1, src, dst)

    sspec = pl.BlockSpec((_NC, _BR, _EMB), lambda i: (0, i, 0))
    cspec = pl.BlockSpec((_NC, _BR, _CNTW), lambda i: (0, i, 0))
    (b2,) = pl.pallas_call(
        _d2_body,
        grid=(_GRID,),
        in_specs=[
            sspec, cspec, _rows(_EMB),
            wfull((_EMB, _EMB)), wfull((1, _EMB)),
            wfull((_EMB, _EMB)), wfull((_EMB, _EMB)), wfull((1, _EMB)),
            wfull((_EMB, _EMB)), wfull((1, _EMB)), wfull((_EMB, _EMB)),
        ],
        out_specs=[_rows(_EMB)],
        out_shape=[bf_shape],
    )(
        s1acc, cnt, ch0,
        fin1W, fin1b, o1t1, o1o1, _row2d(c1["o1_b"]),
        c1["o2_W"], _row2d(c1["o2_b"]), fmr2W,
    )

    s2acc = _edge_pass(a2, b2, dst, src)

    (out,) = pl.pallas_call(
        _d3_body,
        grid=(_GRID,),
        in_specs=[
            sspec, cspec, _rows(_EMB),
            wfull((_EMB, _EMB)), wfull((1, _EMB)),
            wfull((_EMB, _EMB)), wfull((_EMB, _EMB)), wfull((1, _EMB)),
            wfull((_EMB, _EMB)), wfull((1, _EMB)),
            wfull((_EMB, _EMB)), wfull((1, _EMB)),
            wfull((_EMB, 2)), wfull((1, 2)),
        ],
        out_specs=[_rows(2)],
        out_shape=[jax.ShapeDtypeStruct((_N, 2), jnp.float32)],
    )(
        s2acc, cnt, pv0,
        fin2W, fin2b, o1t2, o1o2, _row2d(c2["o1_b"]),
        c2["o2_W"], _row2d(c2["o2_b"]),
        p["out_W1"], _row2d(p["out_b1"]), p["out_W2"], _row2d(p["out_b2"]),
    )
    return out
